# baseline (speedup 1.0000x reference)
import jax
import jax.numpy as jnp
from jax import lax
from jax.experimental import pallas as pl
from jax.experimental.pallas import tpu as pltpu

_B, _T, _D, _V = 32, 8, 1024, 100000
_BT = _B * _T
_V_BLK = 3072
_NB = 3

_SIZES = [1024, 2048] + [3072] * 31 + [1696]
_STARTS = []
_s = 0
for _z in _SIZES:
    _STARTS.append(_s)
    _s += _z
assert _s == _V


def _proj_kernel(
    x_ref, b_ref, w_hbm, o_hbm, xb_ref, wbuf, obuf, tbuf, rsems, wsems
):
    xb_ref[...] = x_ref[...].astype(jnp.bfloat16)
    n = len(_SIZES)

    def _rd(j):
        s, sz = _STARTS[j], _SIZES[j]
        return pltpu.make_async_copy(
            w_hbm.at[pl.ds(s, sz), :],
            wbuf.at[j % _NB, pl.ds(0, sz), :],
            rsems.at[j % _NB],
        )

    def _wr(j):
        s, sz = _STARTS[j], _SIZES[j]
        if j == n - 1:
            return pltpu.make_async_copy(
                tbuf, o_hbm.at[:, pl.ds(s, sz)], wsems.at[2]
            )
        return pltpu.make_async_copy(
            obuf.at[j % 2, :, pl.ds(0, sz)],
            o_hbm.at[:, pl.ds(s, sz)],
            wsems.at[j % 2],
        )

    for j in range(_NB):
        _rd(j).start()
    for j in range(n):
        _rd(j).wait()
        if j >= 2 and j - 2 < n - 1:
            _wr(j - 2).wait()
        s, sz = _STARTS[j], _SIZES[j]
        wb = wbuf[j % _NB, pl.ds(0, sz), :].astype(jnp.bfloat16)
        acc = lax.dot_general(
            xb_ref[...], wb, (((1,), (1,)), ((), ())),
            preferred_element_type=jnp.float32,
        )
        if j == n - 1:
            tbuf[...] = acc + b_ref[:, s : s + sz]
        else:
            obuf[j % 2, :, pl.ds(0, sz)] = acc + b_ref[:, s : s + sz]
        _wr(j).start()
        if j + _NB < n:
            _rd(j + _NB).start()
    _wr(n - 2).wait()
    _wr(n - 1).wait()


def kernel(x, W, b):
    x2 = x.reshape(_BT, _D)
    b2 = b.reshape(1, _V)
    out = pl.pallas_call(
        _proj_kernel,
        grid=(1,),
        in_specs=[
            pl.BlockSpec((_BT, _D), lambda i: (0, 0)),
            pl.BlockSpec((1, _V), lambda i: (0, 0)),
            pl.BlockSpec(memory_space=pltpu.MemorySpace.HBM),
        ],
        out_specs=pl.BlockSpec(memory_space=pltpu.MemorySpace.HBM),
        out_shape=jax.ShapeDtypeStruct((_BT, _V), jnp.float32),
        scratch_shapes=[
            pltpu.VMEM((_BT, _D), jnp.bfloat16),
            pltpu.VMEM((_NB, _V_BLK, _D), jnp.float32),
            pltpu.VMEM((2, _BT, _V_BLK), jnp.float32),
            pltpu.VMEM((_BT, _SIZES[-1]), jnp.float32),
            pltpu.SemaphoreType.DMA((_NB,)),
            pltpu.SemaphoreType.DMA((3,)),
        ],
        compiler_params=pltpu.CompilerParams(
            dimension_semantics=("arbitrary",),
        ),
    )(x2, b2, W)
    return out.reshape(_B, _T, _V)


# speedup vs baseline: 1.0093x; 1.0093x over previous
import jax
import jax.numpy as jnp
from jax.experimental import pallas as pl
from jax.experimental.pallas import tpu as pltpu

_B, _T, _D, _V = 32, 8, 1024, 100000
_BT = _B * _T
_V_BLK = 3200


def _proj_kernel(x_ref, w_ref, b_ref, o_ref):
    xb = x_ref[...].astype(jnp.bfloat16)
    wb = w_ref[...].astype(jnp.bfloat16)
    acc = jax.lax.dot_general(
        xb, wb, (((1,), (1,)), ((), ())), preferred_element_type=jnp.float32
    )
    o_ref[...] = acc + b_ref[...]


def kernel(x, W, b):
    x2 = x.reshape(_BT, _D)
    b2 = b.reshape(1, _V)
    grid = (pl.cdiv(_V, _V_BLK),)
    out = pl.pallas_call(
        _proj_kernel,
        grid=grid,
        in_specs=[
            pl.BlockSpec((_BT, _D), lambda j: (0, 0)),
            pl.BlockSpec((_V_BLK, _D), lambda j: (j, 0)),
            pl.BlockSpec((1, _V_BLK), lambda j: (0, j)),
        ],
        out_specs=pl.BlockSpec((_BT, _V_BLK), lambda j: (0, j)),
        out_shape=jax.ShapeDtypeStruct((_BT, _V), jnp.float32),
        compiler_params=pltpu.CompilerParams(
            dimension_semantics=("arbitrary",),
        ),
    )(x2, W, b2)
    return out.reshape(_B, _T, _V)


# speedup vs baseline: 1.0104x; 1.0010x over previous
import jax
import jax.numpy as jnp
from jax.experimental import pallas as pl
from jax.experimental.pallas import tpu as pltpu

_B, _T, _D, _V = 32, 8, 1024, 100000
_BT = _B * _T
_V_BLK = 2944


def _proj_kernel(x_ref, w_ref, b_ref, o_ref):
    xb = x_ref[...].astype(jnp.bfloat16)
    wb = w_ref[...].astype(jnp.bfloat16)
    acc = jax.lax.dot_general(
        xb, wb, (((1,), (1,)), ((), ())), preferred_element_type=jnp.float32
    )
    o_ref[...] = acc + b_ref[...]


def kernel(x, W, b):
    x2 = x.reshape(_BT, _D)
    b2 = b.reshape(1, _V)
    grid = (pl.cdiv(_V, _V_BLK),)
    out = pl.pallas_call(
        _proj_kernel,
        grid=grid,
        in_specs=[
            pl.BlockSpec((_BT, _D), lambda j: (0, 0)),
            pl.BlockSpec((_V_BLK, _D), lambda j: (j, 0)),
            pl.BlockSpec((1, _V_BLK), lambda j: (0, j)),
        ],
        out_specs=pl.BlockSpec((_BT, _V_BLK), lambda j: (0, j)),
        out_shape=jax.ShapeDtypeStruct((_BT, _V), jnp.float32),
        compiler_params=pltpu.CompilerParams(
            dimension_semantics=("arbitrary",),
        ),
    )(x2, W, b2)
    return out.reshape(_B, _T, _V)
